# baseline (speedup 1.0000x reference)
import functools

import jax
import jax.numpy as jnp
from jax import lax
from jax.experimental import pallas as pl
from jax.experimental.pallas import tpu as pltpu
from jax.experimental.pallas import tpu_sc as plsc

NC = 2
NS = 16
NW = NC * NS
LANES = 16
CHUNK = 80


def _round_up(x, m):
  return (x + m - 1) // m * m


def _make_prep_kernel(n_nodes, n_rel, n_edges):
  rn_pad = _round_up(n_rel * n_nodes, LANES * NW)
  cnt_e = n_edges // NS
  cnt_chunks = cnt_e // CHUNK
  assert cnt_e % CHUNK == 0
  per_tile_rn = rn_pad // NS
  rec_e = n_edges // NW
  rec_chunks = rec_e // CHUNK
  assert rec_e % CHUNK == 0

  mesh = plsc.VectorSubcoreMesh(
      core_axis_name="c", subcore_axis_name="s",
      num_cores=NC, num_subcores=NS)

  @functools.partial(
      pl.kernel,
      out_type=(jax.ShapeDtypeStruct((n_edges,), jnp.int32),
                jax.ShapeDtypeStruct((n_edges,), jnp.float32)),
      mesh=mesh,
      scratch_types=[
          pltpu.VMEM((cnt_e,), jnp.int32),
          pltpu.VMEM((cnt_e,), jnp.int32),
          pltpu.VMEM((rec_e,), jnp.int32),
          pltpu.VMEM((rec_e,), jnp.float32),
          pltpu.VMEM((1, CHUNK), jnp.int32),
          pltpu.VMEM((CHUNK,), jnp.int32),
          pltpu.VMEM((CHUNK,), jnp.float32),
          pltpu.VMEM((per_tile_rn,), jnp.float32),
          pltpu.VMEM_SHARED((rn_pad,), jnp.float32),
      ],
  )
  def prep_kernel(es_hbm, ed_hbm, et_hbm, gidx_hbm, w_hbm,
                  et_v, ed_v, es_v, w_v, idx2_v, idx1_v, cnt_v, zero_v,
                  cnt_sh):
    c = lax.axis_index("c")
    s = lax.axis_index("s")
    w = s * NC + c

    def fill_zero(i, _):
      zero_v[pl.ds(i * LANES, LANES)] = jnp.zeros((LANES,), jnp.float32)
      return 0
    lax.fori_loop(0, per_tile_rn // LANES, fill_zero, 0)
    for v in range(CHUNK // LANES):
      cnt_v[pl.ds(v * LANES, LANES)] = jnp.ones((LANES,), jnp.float32)
    pltpu.sync_copy(zero_v, cnt_sh.at[pl.ds(s * per_tile_rn, per_tile_rn)])
    plsc.subcore_barrier()

    base = s * cnt_e
    pltpu.sync_copy(et_hbm.at[pl.ds(base, cnt_e)], et_v)
    pltpu.sync_copy(ed_hbm.at[pl.ds(base, cnt_e)], ed_v)

    def count_chunk(i, _):
      off = i * CHUNK
      for v in range(CHUNK // LANES):
        t = et_v[pl.ds(off + v * LANES, LANES)]
        d = ed_v[pl.ds(off + v * LANES, LANES)]
        idx2_v[0, pl.ds(v * LANES, LANES)] = t * n_nodes + d
      pltpu.sync_copy(cnt_v, cnt_sh.at[idx2_v.at[0]], add=True)
      return 0
    lax.fori_loop(0, cnt_chunks, count_chunk, 0)
    plsc.subcore_barrier()

    rbase = w * rec_e
    pltpu.sync_copy(et_hbm.at[pl.ds(rbase, rec_e)],
                    et_v.at[pl.ds(0, rec_e)])
    pltpu.sync_copy(ed_hbm.at[pl.ds(rbase, rec_e)],
                    ed_v.at[pl.ds(0, rec_e)])
    pltpu.sync_copy(es_hbm.at[pl.ds(rbase, rec_e)], es_v)

    def rec_chunk(i, _):
      off = i * CHUNK
      for v in range(CHUNK // LANES):
        sl = pl.ds(off + v * LANES, LANES)
        t = et_v[sl]
        d = ed_v[sl]
        sv = es_v[sl]
        idx1_v[pl.ds(v * LANES, LANES)] = t * n_nodes + d
        es_v[sl] = t * n_nodes + sv
      pltpu.sync_copy(cnt_sh.at[idx1_v], cnt_v)
      for v in range(CHUNK // LANES):
        sl = pl.ds(v * LANES, LANES)
        w_v[pl.ds(off + v * LANES, LANES)] = 1.0 / jnp.maximum(cnt_v[sl], 1.0)
      return 0
    lax.fori_loop(0, rec_chunks, rec_chunk, 0)

    pltpu.sync_copy(es_v, gidx_hbm.at[pl.ds(rbase, rec_e)])
    pltpu.sync_copy(w_v, w_hbm.at[pl.ds(rbase, rec_e)])

  return prep_kernel


def _transform(features, weights):
  n, d_in = features.shape
  r, _, d_out = weights.shape
  bn = 2000
  grid = (r, n // bn)

  def body(f_ref, w_ref, h_ref):
    h_ref[0] = jnp.dot(f_ref[...].astype(jnp.bfloat16),
                       w_ref[0].astype(jnp.bfloat16),
                       preferred_element_type=jnp.float32)

  return pl.pallas_call(
      body,
      grid=grid,
      in_specs=[
          pl.BlockSpec((bn, d_in), lambda ri, ni: (ni, 0)),
          pl.BlockSpec((1, d_in, d_out), lambda ri, ni: (ri, 0, 0)),
      ],
      out_specs=pl.BlockSpec((1, bn, d_out), lambda ri, ni: (ri, ni, 0)),
      out_shape=jax.ShapeDtypeStruct((r, n, d_out), jnp.float32),
  )(features, weights)


def _make_main_kernel(n_nodes, n_rel, n_edges, d):
  per_tile_e = n_edges // NW
  n_chunks = per_tile_e // CHUNK
  assert per_tile_e % CHUNK == 0
  phase_a_chunks = (n_chunks * 3) // 5
  phases = [(0, phase_a_chunks), (phase_a_chunks, n_chunks - phase_a_chunks)]
  idx_buf = _round_up(phase_a_chunks * CHUNK, LANES) + LANES
  n_pad = _round_up(n_nodes, 8 * NS)
  rows_pt_pad = n_pad // NS
  dump_full = rows_pt_pad // CHUNK
  dump_tail = rows_pt_pad - dump_full * CHUNK
  assert dump_tail % 8 == 0

  mesh = plsc.VectorSubcoreMesh(
      core_axis_name="c", subcore_axis_name="s",
      num_cores=NC, num_subcores=NS)

  @functools.partial(
      pl.kernel,
      out_type=jax.ShapeDtypeStruct((NC, n_pad, d), jnp.float32),
      mesh=mesh,
      scratch_types=[
          pltpu.VMEM((idx_buf,), jnp.int32),
          pltpu.VMEM((idx_buf,), jnp.int32),
          pltpu.VMEM((idx_buf,), jnp.float32),
          pltpu.VMEM((1, CHUNK), jnp.int32),
          pltpu.VMEM((CHUNK,), jnp.int32),
          pltpu.VMEM((CHUNK,), jnp.int32),
          pltpu.VMEM((CHUNK,), jnp.int32),
          pltpu.VMEM((1, CHUNK), jnp.int32),
          pltpu.VMEM((1, CHUNK), jnp.int32),
          pltpu.VMEM((CHUNK, d), jnp.float32),
          pltpu.VMEM((CHUNK, d), jnp.float32),
          pltpu.VMEM((CHUNK, d), jnp.float32),
          pltpu.VMEM_SHARED((n_pad, d), jnp.float32),
          pltpu.SemaphoreType.DMA,
          pltpu.SemaphoreType.DMA,
          pltpu.SemaphoreType.DMA,
          pltpu.SemaphoreType.DMA,
          pltpu.SemaphoreType.DMA,
          pltpu.SemaphoreType.DMA,
      ],
  )
  def main_kernel(h_hbm, ed_hbm, gidx_hbm, w_hbm, out_hbm,
                  gidx_v, dv_v, w_v, widx0, gidxc0, gidxc1, gidxc2,
                  widx1, widx2, rows0, rows1, rows2, acc_sh,
                  gsem0, gsem1, gsem2, ssem0, ssem1, ssem2):
    c = lax.axis_index("c")
    s = lax.axis_index("s")
    w = s * NC + c
    rows = (rows0, rows1, rows2)
    gidxc = (gidxc0, gidxc1, gidxc2)
    widxc = (widx0, widx1, widx2)
    gsems = (gsem0, gsem1, gsem2)
    ssems = (ssem0, ssem1, ssem2)

    def fill_zero(i, _):
      for v in range(d // LANES):
        rows0[i, pl.ds(v * LANES, LANES)] = jnp.zeros((LANES,), jnp.float32)
      return 0
    lax.fori_loop(0, CHUNK, fill_zero, 0)

    def zero_dma(i, _):
      pltpu.sync_copy(
          rows0, acc_sh.at[pl.ds(s * rows_pt_pad + i * CHUNK, CHUNK)])
      return 0
    lax.fori_loop(0, dump_full, zero_dma, 0)
    if dump_tail:
      pltpu.sync_copy(
          rows0.at[pl.ds(0, dump_tail)],
          acc_sh.at[pl.ds(s * rows_pt_pad + dump_full * CHUNK, dump_tail)])
    plsc.subcore_barrier()

    base = w * per_tile_e

    for ebase, n_c in phases:
      sz = n_c * CHUNK
      pltpu.sync_copy(gidx_hbm.at[pl.ds(base + ebase * CHUNK, sz)],
                      gidx_v.at[pl.ds(0, sz)])
      pltpu.sync_copy(ed_hbm.at[pl.ds(base + ebase * CHUNK, sz)],
                      dv_v.at[pl.ds(0, sz)])
      pltpu.sync_copy(w_hbm.at[pl.ds(base + ebase * CHUNK, sz)],
                      w_v.at[pl.ds(0, sz)])

      def stage_idx(cd, slot):
        for v in range(CHUNK // LANES):
          sl = pl.ds(v * LANES, LANES)
          gidxc[slot][sl] = gidx_v[pl.ds(cd * CHUNK + v * LANES, LANES)]
          widxc[slot][0, sl] = dv_v[pl.ds(cd * CHUNK + v * LANES, LANES)]

      def issue_gather(slot):
        pltpu.async_copy(h_hbm.at[gidxc[slot]], rows[slot], gsems[slot])

      def wait_gather(slot):
        pltpu.make_async_copy(
            h_hbm.at[pl.ds(0, CHUNK)], rows[slot], gsems[slot]).wait()

      def issue_scatter(slot):
        pltpu.async_copy(rows[slot], acc_sh.at[widxc[slot].at[0]],
                         ssems[slot], add=True)

      def wait_scatter(slot):
        pltpu.make_async_copy(
            rows[slot], acc_sh.at[pl.ds(0, CHUNK)], ssems[slot]).wait()

      zidx = jnp.zeros((LANES, 1), jnp.int32)
      dnums = lax.GatherDimensionNumbers(
          offset_dims=(), collapsed_slice_dims=(0,), start_index_map=(0,))

      def scale(cd, slot):
        rbuf = rows[slot]
        def edge(e, _):
          wvec = w_v[pl.ds(cd * CHUNK + e, LANES)]
          wsplat = lax.gather(wvec, zidx, dnums, slice_sizes=(1,),
                              mode=lax.GatherScatterMode.PROMISE_IN_BOUNDS)
          for k in range(d // LANES):
            sl = pl.ds(k * LANES, LANES)
            rbuf[e, sl] = rbuf[e, sl] * wsplat
          return 0
        lax.fori_loop(0, CHUNK, edge, 0, unroll=4)

      def subblock(cd, slot, static_c=None):
        wait_gather(slot)
        scale(cd, slot)
        nslot = (slot + 2) % 3
        if static_c is None:
          @pl.when(cd >= 1)
          def _():
            wait_scatter(nslot)
          @pl.when(cd + 2 < n_c)
          def _():
            stage_idx(cd + 2, nslot)
            issue_gather(nslot)
        else:
          if static_c >= 1:
            wait_scatter(nslot)
          if static_c + 2 < n_c:
            stage_idx(cd + 2, nslot)
            issue_gather(nslot)
        issue_scatter(slot)

      stage_idx(0, 0)
      issue_gather(0)
      stage_idx(1, 1)
      issue_gather(1)
      n_full = n_c // 3
      rem = n_c - n_full * 3

      def body(j, _):
        cd = j * 3
        subblock(cd, 0)
        subblock(cd + 1, 1)
        subblock(cd + 2, 2)
        return 0
      lax.fori_loop(0, n_full, body, 0)
      for t in range(rem):
        cd = n_full * 3 + t
        subblock(cd, t, static_c=cd)
      wait_scatter((n_c - 1) % 3)

    plsc.subcore_barrier()
    def dump(j, _):
      off = s * rows_pt_pad + j * CHUNK
      pltpu.sync_copy(acc_sh.at[pl.ds(off, CHUNK)], rows0)
      pltpu.sync_copy(rows0, out_hbm.at[c, pl.ds(off, CHUNK)])
      return 0
    lax.fori_loop(0, dump_full, dump, 0)
    if dump_tail:
      off = s * rows_pt_pad + dump_full * CHUNK
      pltpu.sync_copy(acc_sh.at[pl.ds(off, dump_tail)],
                      rows0.at[pl.ds(0, dump_tail)])
      pltpu.sync_copy(rows0.at[pl.ds(0, dump_tail)],
                      out_hbm.at[c, pl.ds(off, dump_tail)])

  return main_kernel, n_pad


def _combine(partial, bias2d, n):
  nc, _, d = partial.shape
  bn = 2000
  grid = (n // bn,)

  def body(p_ref, b_ref, o_ref):
    o_ref[...] = p_ref[0] + p_ref[1] + b_ref[...]

  return pl.pallas_call(
      body,
      grid=grid,
      in_specs=[
          pl.BlockSpec((nc, bn, d), lambda i: (0, i, 0)),
          pl.BlockSpec((1, d), lambda i: (0, 0)),
      ],
      out_specs=pl.BlockSpec((bn, d), lambda i: (i, 0)),
      out_shape=jax.ShapeDtypeStruct((n, d), jnp.float32),
  )(partial, bias2d)


def kernel(features, edge_index, edge_type, weights, bias):
  n, d_in = features.shape
  r, _, d_out = weights.shape
  e = edge_index.shape[1]
  src = edge_index[0]
  dst = edge_index[1]
  et = edge_type.astype(jnp.int32)

  prep_kernel = _make_prep_kernel(n, r, e)
  gidx, w = prep_kernel(src, dst, et)

  h = _transform(features, weights).reshape(r * n, d_out)

  main_kernel, n_pad = _make_main_kernel(n, r, e, d_out)
  partial = main_kernel(h, dst, gidx, w)

  return _combine(partial, bias.reshape(1, d_out), n)


# speedup vs baseline: 1.8633x; 1.8633x over previous
import functools

import jax
import jax.numpy as jnp
from jax import lax
from jax.experimental import pallas as pl
from jax.experimental.pallas import tpu as pltpu
from jax.experimental.pallas import tpu_sc as plsc

NC = 2
NS = 16
NW = NC * NS
LANES = 16
CHUNK = 80


def _round_up(x, m):
  return (x + m - 1) // m * m


def _make_prep_kernel(n_nodes, n_rel, n_edges):
  rn_pad = _round_up(n_rel * n_nodes, LANES * NW)
  cnt_e = n_edges // NS
  cnt_chunks = cnt_e // CHUNK
  assert cnt_e % CHUNK == 0
  per_tile_rn = rn_pad // NS
  rec_e = n_edges // NW
  rec_chunks = rec_e // CHUNK
  assert rec_e % CHUNK == 0

  mesh = plsc.VectorSubcoreMesh(
      core_axis_name="c", subcore_axis_name="s",
      num_cores=NC, num_subcores=NS)

  @functools.partial(
      pl.kernel,
      out_type=(jax.ShapeDtypeStruct((n_edges,), jnp.int32),
                jax.ShapeDtypeStruct((n_edges,), jnp.float32)),
      mesh=mesh,
      scratch_types=[
          pltpu.VMEM((cnt_e,), jnp.int32),
          pltpu.VMEM((cnt_e,), jnp.int32),
          pltpu.VMEM((rec_e,), jnp.int32),
          pltpu.VMEM((rec_e,), jnp.float32),
          pltpu.VMEM((1, CHUNK), jnp.int32),
          pltpu.VMEM((CHUNK,), jnp.int32),
          pltpu.VMEM((CHUNK,), jnp.float32),
          pltpu.VMEM((per_tile_rn,), jnp.float32),
          pltpu.VMEM_SHARED((rn_pad,), jnp.float32),
      ],
  )
  def prep_kernel(es_hbm, ed_hbm, et_hbm, gidx_hbm, w_hbm,
                  et_v, ed_v, es_v, w_v, idx2_v, idx1_v, cnt_v, zero_v,
                  cnt_sh):
    c = lax.axis_index("c")
    s = lax.axis_index("s")
    w = s * NC + c

    def fill_zero(i, _):
      zero_v[pl.ds(i * LANES, LANES)] = jnp.zeros((LANES,), jnp.float32)
      return 0
    lax.fori_loop(0, per_tile_rn // LANES, fill_zero, 0)
    for v in range(CHUNK // LANES):
      cnt_v[pl.ds(v * LANES, LANES)] = jnp.ones((LANES,), jnp.float32)
    pltpu.sync_copy(zero_v, cnt_sh.at[pl.ds(s * per_tile_rn, per_tile_rn)])
    plsc.subcore_barrier()

    base = s * cnt_e
    pltpu.sync_copy(et_hbm.at[pl.ds(base, cnt_e)], et_v)
    pltpu.sync_copy(ed_hbm.at[pl.ds(base, cnt_e)], ed_v)

    def count_chunk(i, _):
      off = i * CHUNK
      for v in range(CHUNK // LANES):
        t = et_v[pl.ds(off + v * LANES, LANES)]
        d = ed_v[pl.ds(off + v * LANES, LANES)]
        idx2_v[0, pl.ds(v * LANES, LANES)] = t * n_nodes + d
      pltpu.sync_copy(cnt_v, cnt_sh.at[idx2_v.at[0]], add=True)
      return 0
    lax.fori_loop(0, cnt_chunks, count_chunk, 0)
    plsc.subcore_barrier()

    rbase = w * rec_e
    pltpu.sync_copy(et_hbm.at[pl.ds(rbase, rec_e)],
                    et_v.at[pl.ds(0, rec_e)])
    pltpu.sync_copy(ed_hbm.at[pl.ds(rbase, rec_e)],
                    ed_v.at[pl.ds(0, rec_e)])
    pltpu.sync_copy(es_hbm.at[pl.ds(rbase, rec_e)], es_v)

    def rec_chunk(i, _):
      off = i * CHUNK
      for v in range(CHUNK // LANES):
        sl = pl.ds(off + v * LANES, LANES)
        t = et_v[sl]
        d = ed_v[sl]
        sv = es_v[sl]
        idx1_v[pl.ds(v * LANES, LANES)] = t * n_nodes + d
        es_v[sl] = t * n_nodes + sv
      pltpu.sync_copy(cnt_sh.at[idx1_v], cnt_v)
      for v in range(CHUNK // LANES):
        sl = pl.ds(v * LANES, LANES)
        w_v[pl.ds(off + v * LANES, LANES)] = 1.0 / jnp.maximum(cnt_v[sl], 1.0)
      return 0
    lax.fori_loop(0, rec_chunks, rec_chunk, 0)

    pltpu.sync_copy(es_v, gidx_hbm.at[pl.ds(rbase, rec_e)])
    pltpu.sync_copy(w_v, w_hbm.at[pl.ds(rbase, rec_e)])

  return prep_kernel


def _transform(features, weights):
  n, d_in = features.shape
  r, _, d_out = weights.shape
  bn = 2000
  grid = (r, n // bn)

  def body(f_ref, w_ref, h_ref):
    h_ref[0] = jnp.dot(f_ref[...].astype(jnp.bfloat16),
                       w_ref[0].astype(jnp.bfloat16),
                       preferred_element_type=jnp.float32)

  return pl.pallas_call(
      body,
      grid=grid,
      in_specs=[
          pl.BlockSpec((bn, d_in), lambda ri, ni: (ni, 0)),
          pl.BlockSpec((1, d_in, d_out), lambda ri, ni: (ri, 0, 0)),
      ],
      out_specs=pl.BlockSpec((1, bn, d_out), lambda ri, ni: (ri, ni, 0)),
      out_shape=jax.ShapeDtypeStruct((r, n, d_out), jnp.float32),
  )(features, weights)


def _make_main_kernel(n_nodes, n_rel, n_edges, d):
  per_tile_e = n_edges // NW
  n_chunks = per_tile_e // CHUNK
  assert per_tile_e % CHUNK == 0
  phase_a_chunks = (n_chunks * 3) // 5
  phases = [(0, phase_a_chunks), (phase_a_chunks, n_chunks - phase_a_chunks)]
  idx_buf = _round_up(phase_a_chunks * CHUNK, LANES) + LANES
  n_pad = _round_up(n_nodes, 8 * NS)
  rows_pt_pad = n_pad // NS
  dump_full = rows_pt_pad // CHUNK
  dump_tail = rows_pt_pad - dump_full * CHUNK
  assert dump_tail % 8 == 0

  mesh = plsc.VectorSubcoreMesh(
      core_axis_name="c", subcore_axis_name="s",
      num_cores=NC, num_subcores=NS)

  @functools.partial(
      pl.kernel,
      out_type=jax.ShapeDtypeStruct((NC, n_pad, d), jnp.float32),
      mesh=mesh,
      scratch_types=[
          pltpu.VMEM((idx_buf,), jnp.int32),
          pltpu.VMEM((idx_buf,), jnp.int32),
          pltpu.VMEM((idx_buf,), jnp.float32),
          pltpu.VMEM((1, CHUNK), jnp.int32),
          pltpu.VMEM((CHUNK,), jnp.int32),
          pltpu.VMEM((CHUNK,), jnp.int32),
          pltpu.VMEM((CHUNK,), jnp.int32),
          pltpu.VMEM((1, CHUNK), jnp.int32),
          pltpu.VMEM((1, CHUNK), jnp.int32),
          pltpu.VMEM((CHUNK, d), jnp.float32),
          pltpu.VMEM((CHUNK, d), jnp.float32),
          pltpu.VMEM((CHUNK, d), jnp.float32),
          pltpu.VMEM_SHARED((n_pad, d), jnp.float32),
          pltpu.SemaphoreType.DMA,
          pltpu.SemaphoreType.DMA,
          pltpu.SemaphoreType.DMA,
          pltpu.SemaphoreType.DMA,
          pltpu.SemaphoreType.DMA,
          pltpu.SemaphoreType.DMA,
      ],
  )
  def main_kernel(h_hbm, ed_hbm, gidx_hbm, w_hbm, out_hbm,
                  gidx_v, dv_v, w_v, widx0, gidxc0, gidxc1, gidxc2,
                  widx1, widx2, rows0, rows1, rows2, acc_sh,
                  gsem0, gsem1, gsem2, ssem0, ssem1, ssem2):
    c = lax.axis_index("c")
    s = lax.axis_index("s")
    w = s * NC + c
    rows = (rows0, rows1, rows2)
    gidxc = (gidxc0, gidxc1, gidxc2)
    widxc = (widx0, widx1, widx2)
    gsems = (gsem0, gsem1, gsem2)
    ssems = (ssem0, ssem1, ssem2)

    def fill_zero(i, _):
      for v in range(d // LANES):
        rows0[i, pl.ds(v * LANES, LANES)] = jnp.zeros((LANES,), jnp.float32)
      return 0
    lax.fori_loop(0, CHUNK, fill_zero, 0)

    def zero_dma(i, _):
      pltpu.sync_copy(
          rows0, acc_sh.at[pl.ds(s * rows_pt_pad + i * CHUNK, CHUNK)])
      return 0
    lax.fori_loop(0, dump_full, zero_dma, 0)
    if dump_tail:
      pltpu.sync_copy(
          rows0.at[pl.ds(0, dump_tail)],
          acc_sh.at[pl.ds(s * rows_pt_pad + dump_full * CHUNK, dump_tail)])
    plsc.subcore_barrier()

    base = w * per_tile_e

    for ebase, n_c in phases:
      sz = n_c * CHUNK
      pltpu.sync_copy(gidx_hbm.at[pl.ds(base + ebase * CHUNK, sz)],
                      gidx_v.at[pl.ds(0, sz)])
      pltpu.sync_copy(ed_hbm.at[pl.ds(base + ebase * CHUNK, sz)],
                      dv_v.at[pl.ds(0, sz)])
      pltpu.sync_copy(w_hbm.at[pl.ds(base + ebase * CHUNK, sz)],
                      w_v.at[pl.ds(0, sz)])

      def stage_idx(cd, slot):
        for v in range(CHUNK // LANES):
          sl = pl.ds(v * LANES, LANES)
          gidxc[slot][sl] = gidx_v[pl.ds(cd * CHUNK + v * LANES, LANES)]
          widxc[slot][0, sl] = dv_v[pl.ds(cd * CHUNK + v * LANES, LANES)]

      def issue_gather(slot):
        pltpu.async_copy(h_hbm.at[gidxc[slot]], rows[slot], gsems[slot])

      def wait_gather(slot):
        pltpu.make_async_copy(
            h_hbm.at[pl.ds(0, CHUNK)], rows[slot], gsems[slot]).wait()

      def issue_scatter(slot):
        pltpu.async_copy(rows[slot], acc_sh.at[widxc[slot].at[0]],
                         ssems[slot], add=True)

      def wait_scatter(slot):
        pltpu.make_async_copy(
            rows[slot], acc_sh.at[pl.ds(0, CHUNK)], ssems[slot]).wait()

      def scale(cd, slot):
        rbuf = rows[slot]
        def edge(e, _):
          wvec = w_v[pl.ds(cd * CHUNK + e, LANES)]
          winv = wvec[0]
          for k in range(d // LANES):
            sl = pl.ds(k * LANES, LANES)
            rbuf[e, sl] = rbuf[e, sl] * winv
          return 0
        lax.fori_loop(0, CHUNK, edge, 0, unroll=4)

      def subblock(cd, slot, static_c=None):
        wait_gather(slot)
        scale(cd, slot)
        nslot = (slot + 2) % 3
        if static_c is None:
          @pl.when(cd >= 1)
          def _():
            wait_scatter(nslot)
          @pl.when(cd + 2 < n_c)
          def _():
            stage_idx(cd + 2, nslot)
            issue_gather(nslot)
        else:
          if static_c >= 1:
            wait_scatter(nslot)
          if static_c + 2 < n_c:
            stage_idx(cd + 2, nslot)
            issue_gather(nslot)
        issue_scatter(slot)

      stage_idx(0, 0)
      issue_gather(0)
      stage_idx(1, 1)
      issue_gather(1)
      n_full = n_c // 3
      rem = n_c - n_full * 3

      def body(j, _):
        cd = j * 3
        subblock(cd, 0)
        subblock(cd + 1, 1)
        subblock(cd + 2, 2)
        return 0
      lax.fori_loop(0, n_full, body, 0)
      for t in range(rem):
        cd = n_full * 3 + t
        subblock(cd, t, static_c=cd)
      wait_scatter((n_c - 1) % 3)

    plsc.subcore_barrier()
    def dump(j, _):
      off = s * rows_pt_pad + j * CHUNK
      pltpu.sync_copy(acc_sh.at[pl.ds(off, CHUNK)], rows0)
      pltpu.sync_copy(rows0, out_hbm.at[c, pl.ds(off, CHUNK)])
      return 0
    lax.fori_loop(0, dump_full, dump, 0)
    if dump_tail:
      off = s * rows_pt_pad + dump_full * CHUNK
      pltpu.sync_copy(acc_sh.at[pl.ds(off, dump_tail)],
                      rows0.at[pl.ds(0, dump_tail)])
      pltpu.sync_copy(rows0.at[pl.ds(0, dump_tail)],
                      out_hbm.at[c, pl.ds(off, dump_tail)])

  return main_kernel, n_pad


def _combine(partial, bias2d, n):
  nc, _, d = partial.shape
  bn = 2000
  grid = (n // bn,)

  def body(p_ref, b_ref, o_ref):
    o_ref[...] = p_ref[0] + p_ref[1] + b_ref[...]

  return pl.pallas_call(
      body,
      grid=grid,
      in_specs=[
          pl.BlockSpec((nc, bn, d), lambda i: (0, i, 0)),
          pl.BlockSpec((1, d), lambda i: (0, 0)),
      ],
      out_specs=pl.BlockSpec((bn, d), lambda i: (i, 0)),
      out_shape=jax.ShapeDtypeStruct((n, d), jnp.float32),
  )(partial, bias2d)


def kernel(features, edge_index, edge_type, weights, bias):
  n, d_in = features.shape
  r, _, d_out = weights.shape
  e = edge_index.shape[1]
  src = edge_index[0]
  dst = edge_index[1]
  et = edge_type.astype(jnp.int32)

  prep_kernel = _make_prep_kernel(n, r, e)
  gidx, w = prep_kernel(src, dst, et)

  h = _transform(features, weights).reshape(r * n, d_out)

  main_kernel, n_pad = _make_main_kernel(n, r, e, d_out)
  partial = main_kernel(h, dst, gidx, w)

  return _combine(partial, bias.reshape(1, d_out), n)
